# Initial kernel scaffold; baseline (speedup 1.0000x reference)
import jax
import jax.numpy as jnp
from jax.experimental import pallas as pl


def kernel(x, edge_index, W1l, W1r, b1, W2l, W2r, b2, W3l, W3r, b3):
    raise NotImplementedError("write your pallas kernel here")


# speedup vs baseline: 3.4874x; 3.4874x over previous
import functools

import jax
import jax.numpy as jnp
from jax import lax
from jax.experimental import pallas as pl
from jax.experimental.pallas import tpu as pltpu
from jax.experimental.pallas import tpu_sc as plsc

_N = 50000
_NPAD = 51200
_C = 32
_BATCH = 128
_NW = 32
_NBW = 391
_NBTOT = _NW * _NBW
_EPAD = _NBTOT * _BATCH
_ZROWS = 128
_R = 1024


def _sc_agg(table, srcb, dstb):
    mesh = plsc.VectorSubcoreMesh(core_axis_name="c", subcore_axis_name="s")
    rows_per_sub = _NPAD // 16

    @functools.partial(
        pl.kernel,
        out_type=jax.ShapeDtypeStruct((2, _NPAD, _C), jnp.float32),
        mesh=mesh,
        compiler_params=pltpu.CompilerParams(use_tc_tiling_on_sc=False),
        scratch_types=[
            pltpu.VMEM((_BATCH,), jnp.int32),
            pltpu.VMEM((_BATCH,), jnp.int32),
            pltpu.VMEM((_BATCH, _C), jnp.float32),
            pltpu.VMEM((_ZROWS, _C), jnp.float32),
            pltpu.VMEM_SHARED((_NPAD, _C), jnp.float32),
            pltpu.SemaphoreType.DMA,
        ],
    )
    def k(table_hbm, srcb_hbm, dstb_hbm, out_hbm, src_v, dst_v, rows_v, zbuf,
          acc, sem):
        cid = lax.axis_index("c")
        sid = lax.axis_index("s")
        wid = cid * 16 + sid

        def zb(i, carry):
            zbuf[i, pl.ds(0, 16)] = jnp.zeros((16,), jnp.float32)
            zbuf[i, pl.ds(16, 16)] = jnp.zeros((16,), jnp.float32)
            return carry

        lax.fori_loop(0, _ZROWS, zb, 0)

        def zc(i, carry):
            pltpu.sync_copy(
                zbuf, acc.at[pl.ds(sid * rows_per_sub + i * _ZROWS, _ZROWS)])
            return carry

        lax.fori_loop(0, rows_per_sub // _ZROWS, zc, 0)
        plsc.subcore_barrier()

        base = wid * _NBW

        def body(j, carry):
            pltpu.sync_copy(srcb_hbm.at[base + j], src_v)
            pltpu.sync_copy(dstb_hbm.at[base + j], dst_v)
            pltpu.async_copy(table_hbm.at[src_v], rows_v, sem).wait()
            pltpu.sync_copy(rows_v, acc.at[dst_v], add=True)
            return carry

        lax.fori_loop(0, _NBW, body, 0)
        plsc.subcore_barrier()

        def wb(i, carry):
            off = sid * rows_per_sub + i * 640
            pltpu.sync_copy(acc.at[pl.ds(off, 640)],
                            out_hbm.at[cid].at[pl.ds(off, 640)])
            return carry

        lax.fori_loop(0, rows_per_sub // 640, wb, 0)

    return k(table, srcb, dstb)


def _tc_layer1(part, x_pad, wlt, wrt, b):
    def body(p_ref, x_ref, wl_ref, wr_ref, b_ref, s0, s1, s2, s3, inv_ref):
        p = p_ref[0] + p_ref[1]
        cnt = p[:, 26:27]
        inv = 1.0 / jnp.maximum(cnt, 1.0)
        h = (inv * jnp.dot(p, wl_ref[...], preferred_element_type=jnp.float32)
             + jnp.dot(x_ref[...], wr_ref[...],
                       preferred_element_type=jnp.float32)
             + b_ref[...])
        h = jnp.maximum(h, 0.0)
        s0[...] = h[:, 0:32]
        s1[...] = h[:, 32:64]
        s2[...] = h[:, 64:96]
        s3[...] = h[:, 96:128]
        inv_ref[...] = inv

    slab = jax.ShapeDtypeStruct((_NPAD, _C), jnp.float32)
    return pl.pallas_call(
        body,
        grid=(_NPAD // _R,),
        in_specs=[
            pl.BlockSpec((2, _R, _C), lambda i: (0, i, 0)),
            pl.BlockSpec((_R, _C), lambda i: (i, 0)),
            pl.BlockSpec((_C, 128), lambda i: (0, 0)),
            pl.BlockSpec((_C, 128), lambda i: (0, 0)),
            pl.BlockSpec((1, 128), lambda i: (0, 0)),
        ],
        out_specs=[
            pl.BlockSpec((_R, _C), lambda i: (i, 0)),
            pl.BlockSpec((_R, _C), lambda i: (i, 0)),
            pl.BlockSpec((_R, _C), lambda i: (i, 0)),
            pl.BlockSpec((_R, _C), lambda i: (i, 0)),
            pl.BlockSpec((_R, 1), lambda i: (i, 0)),
        ],
        out_shape=[slab, slab, slab, slab,
                   jax.ShapeDtypeStruct((_NPAD, 1), jnp.float32)],
    )(part, x_pad, wlt, wrt, b)


def _tc_layer(parts, slabs, inv, wlt, wrt, b, final):
    def body(p0, p1, p2, p3, s0, s1, s2, s3, inv_ref, wl_ref, wr_ref, b_ref,
             *outs):
        h = b_ref[...] + jnp.zeros((_R, 128), jnp.float32)
        agg_mm = jnp.zeros((_R, 128), jnp.float32)
        for c, (p_ref, s_ref) in enumerate(
                zip((p0, p1, p2, p3), (s0, s1, s2, s3))):
            agg_mm = agg_mm + jnp.dot(p_ref[0] + p_ref[1],
                                      wl_ref[pl.ds(c * _C, _C), :],
                                      preferred_element_type=jnp.float32)
            h = h + jnp.dot(s_ref[...], wr_ref[pl.ds(c * _C, _C), :],
                            preferred_element_type=jnp.float32)
        h = jnp.maximum(inv_ref[...] * agg_mm + h, 0.0)
        if final:
            outs[0][...] = h
        else:
            for c in range(4):
                outs[c][...] = h[:, c * _C:(c + 1) * _C]

    part_spec = pl.BlockSpec((2, _R, _C), lambda i: (0, i, 0))
    slab_spec = pl.BlockSpec((_R, _C), lambda i: (i, 0))
    w_spec = pl.BlockSpec((128, 128), lambda i: (0, 0))
    if final:
        grid = ((_N + _R - 1) // _R,)
        out_specs = [pl.BlockSpec((_R, 128), lambda i: (i, 0))]
        out_shape = [jax.ShapeDtypeStruct((_N, 128), jnp.float32)]
    else:
        grid = (_NPAD // _R,)
        out_specs = [slab_spec] * 4
        out_shape = [jax.ShapeDtypeStruct((_NPAD, _C), jnp.float32)] * 4
    out = pl.pallas_call(
        body,
        grid=grid,
        in_specs=[part_spec] * 4 + [slab_spec] * 4 + [
            pl.BlockSpec((_R, 1), lambda i: (i, 0)), w_spec, w_spec,
            pl.BlockSpec((1, 128), lambda i: (0, 0)),
        ],
        out_specs=out_specs,
        out_shape=out_shape,
    )(*parts, *slabs, inv, wlt, wrt, b)
    return out[0] if final else out


def kernel(x, edge_index, W1l, W1r, b1, W2l, W2r, b2, W3l, W3r, b3):
    e = edge_index.shape[1]
    src = edge_index[0].astype(jnp.int32)
    dst = edge_index[1].astype(jnp.int32)
    pad = jnp.full((_EPAD - e,), _N, jnp.int32)
    srcb = jnp.concatenate([src, pad]).reshape(_NBTOT, _BATCH)
    dstb = jnp.concatenate([dst, pad]).reshape(_NBTOT, _BATCH)

    x_pad = jnp.zeros((_NPAD, _C), jnp.float32)
    x_pad = x_pad.at[:_N, :26].set(x).at[:_N, 26].set(1.0)

    w1lt = jnp.zeros((_C, 128), jnp.float32).at[:26].set(W1l.T)
    w1rt = jnp.zeros((_C, 128), jnp.float32).at[:26].set(W1r.T)

    part1 = _sc_agg(x_pad, srcb, dstb)
    *slabs1, inv = _tc_layer1(part1, x_pad, w1lt, w1rt, b1.reshape(1, 128))

    parts2 = [_sc_agg(slabs1[c], srcb, dstb) for c in range(4)]
    slabs2 = _tc_layer(parts2, slabs1, inv, W2l.T, W2r.T,
                       b2.reshape(1, 128), final=False)

    parts3 = [_sc_agg(slabs2[c], srcb, dstb) for c in range(4)]
    return _tc_layer(parts3, slabs2, inv, W3l.T, W3r.T,
                     b3.reshape(1, 128), final=True)


# speedup vs baseline: 5.6856x; 1.6303x over previous
import functools

import jax
import jax.numpy as jnp
from jax import lax
from jax.experimental import pallas as pl
from jax.experimental.pallas import tpu as pltpu
from jax.experimental.pallas import tpu_sc as plsc

_N = 50000
_NPAD = 51200
_C = 32
_BATCH = 128
_NW = 32
_K = 3
_G = 132
_NBW = _K * _G
_NBTOT = _NW * _NBW
_EPAD = _NBTOT * _BATCH
_ZROWS = 64
_R = 1024


def _sc_agg(table, srcb, dstb):
    mesh = plsc.VectorSubcoreMesh(core_axis_name="c", subcore_axis_name="s")
    rows_per_sub = _NPAD // 16

    @functools.partial(
        pl.kernel,
        out_type=jax.ShapeDtypeStruct((2, _NPAD, _C), jnp.float32),
        mesh=mesh,
        compiler_params=pltpu.CompilerParams(use_tc_tiling_on_sc=False),
        scratch_types=[
            pltpu.VMEM((2, _K, _BATCH), jnp.int32),
            pltpu.VMEM((2, _K, _BATCH), jnp.int32),
            pltpu.VMEM((2, _K, _BATCH, _C), jnp.float32),
            pltpu.VMEM((_ZROWS, _C), jnp.float32),
            pltpu.VMEM_SHARED((_NPAD, _C), jnp.float32),
            pltpu.SemaphoreType.DMA,
            pltpu.SemaphoreType.DMA,
        ],
    )
    def k(table_hbm, srcb_hbm, dstb_hbm, out_hbm, src_v, dst_v, rows_v, zbuf,
          acc, sem0, sem1):
        cid = lax.axis_index("c")
        sid = lax.axis_index("s")
        wid = cid * 16 + sid
        sems = (sem0, sem1)

        def zb(i, carry):
            zbuf[i, pl.ds(0, 16)] = jnp.zeros((16,), jnp.float32)
            zbuf[i, pl.ds(16, 16)] = jnp.zeros((16,), jnp.float32)
            return carry

        lax.fori_loop(0, _ZROWS, zb, 0)

        def zc(i, carry):
            pltpu.sync_copy(
                zbuf, acc.at[pl.ds(sid * rows_per_sub + i * _ZROWS, _ZROWS)])
            return carry

        lax.fori_loop(0, rows_per_sub // _ZROWS, zc, 0)
        plsc.subcore_barrier()

        base = wid * _NBW

        def fire(g, b):
            row = base + g * _K
            pltpu.sync_copy(srcb_hbm.at[pl.ds(row, _K)], src_v.at[b])
            pltpu.sync_copy(dstb_hbm.at[pl.ds(row, _K)], dst_v.at[b])
            for j in range(_K):
                pltpu.async_copy(table_hbm.at[src_v.at[b, j]],
                                 rows_v.at[b, j], sems[b])

        def drain_scatter(b):
            for j in range(_K):
                pltpu.make_async_copy(table_hbm.at[src_v.at[b, j]],
                                      rows_v.at[b, j], sems[b]).wait()
            for j in range(_K):
                pltpu.sync_copy(rows_v.at[b, j], acc.at[dst_v.at[b, j]],
                                add=True)

        fire(0, 0)

        def body(i, carry):
            g = 2 * i
            fire(g + 1, 1)
            drain_scatter(0)

            @pl.when(g + 2 < _G)
            def _():
                fire(g + 2, 0)

            drain_scatter(1)
            return carry

        lax.fori_loop(0, _G // 2, body, 0)
        plsc.subcore_barrier()

        def wb(i, carry):
            off = sid * rows_per_sub + i * 640
            pltpu.sync_copy(acc.at[pl.ds(off, 640)],
                            out_hbm.at[cid].at[pl.ds(off, 640)])
            return carry

        lax.fori_loop(0, rows_per_sub // 640, wb, 0)

    return k(table, srcb, dstb)


def _tc_layer1(part, x_pad, wlt, wrt, b):
    def body(p_ref, x_ref, wl_ref, wr_ref, b_ref, s0, s1, s2, s3, inv_ref):
        p = p_ref[0] + p_ref[1]
        cnt = p[:, 26:27]
        inv = 1.0 / jnp.maximum(cnt, 1.0)
        h = (inv * jnp.dot(p, wl_ref[...], preferred_element_type=jnp.float32)
             + jnp.dot(x_ref[...], wr_ref[...],
                       preferred_element_type=jnp.float32)
             + b_ref[...])
        h = jnp.maximum(h, 0.0)
        s0[...] = h[:, 0:32]
        s1[...] = h[:, 32:64]
        s2[...] = h[:, 64:96]
        s3[...] = h[:, 96:128]
        inv_ref[...] = inv

    slab = jax.ShapeDtypeStruct((_NPAD, _C), jnp.float32)
    return pl.pallas_call(
        body,
        grid=(_NPAD // _R,),
        in_specs=[
            pl.BlockSpec((2, _R, _C), lambda i: (0, i, 0)),
            pl.BlockSpec((_R, _C), lambda i: (i, 0)),
            pl.BlockSpec((_C, 128), lambda i: (0, 0)),
            pl.BlockSpec((_C, 128), lambda i: (0, 0)),
            pl.BlockSpec((1, 128), lambda i: (0, 0)),
        ],
        out_specs=[
            pl.BlockSpec((_R, _C), lambda i: (i, 0)),
            pl.BlockSpec((_R, _C), lambda i: (i, 0)),
            pl.BlockSpec((_R, _C), lambda i: (i, 0)),
            pl.BlockSpec((_R, _C), lambda i: (i, 0)),
            pl.BlockSpec((_R, 1), lambda i: (i, 0)),
        ],
        out_shape=[slab, slab, slab, slab,
                   jax.ShapeDtypeStruct((_NPAD, 1), jnp.float32)],
    )(part, x_pad, wlt, wrt, b)


def _tc_layer(parts, slabs, inv, wlt, wrt, b, final):
    def body(p0, p1, p2, p3, s0, s1, s2, s3, inv_ref, wl_ref, wr_ref, b_ref,
             *outs):
        h = b_ref[...] + jnp.zeros((_R, 128), jnp.float32)
        agg_mm = jnp.zeros((_R, 128), jnp.float32)
        for c, (p_ref, s_ref) in enumerate(
                zip((p0, p1, p2, p3), (s0, s1, s2, s3))):
            agg_mm = agg_mm + jnp.dot(p_ref[0] + p_ref[1],
                                      wl_ref[pl.ds(c * _C, _C), :],
                                      preferred_element_type=jnp.float32)
            h = h + jnp.dot(s_ref[...], wr_ref[pl.ds(c * _C, _C), :],
                            preferred_element_type=jnp.float32)
        h = jnp.maximum(inv_ref[...] * agg_mm + h, 0.0)
        if final:
            outs[0][...] = h
        else:
            for c in range(4):
                outs[c][...] = h[:, c * _C:(c + 1) * _C]

    part_spec = pl.BlockSpec((2, _R, _C), lambda i: (0, i, 0))
    slab_spec = pl.BlockSpec((_R, _C), lambda i: (i, 0))
    w_spec = pl.BlockSpec((128, 128), lambda i: (0, 0))
    if final:
        grid = ((_N + _R - 1) // _R,)
        out_specs = [pl.BlockSpec((_R, 128), lambda i: (i, 0))]
        out_shape = [jax.ShapeDtypeStruct((_N, 128), jnp.float32)]
    else:
        grid = (_NPAD // _R,)
        out_specs = [slab_spec] * 4
        out_shape = [jax.ShapeDtypeStruct((_NPAD, _C), jnp.float32)] * 4
    out = pl.pallas_call(
        body,
        grid=grid,
        in_specs=[part_spec] * 4 + [slab_spec] * 4 + [
            pl.BlockSpec((_R, 1), lambda i: (i, 0)), w_spec, w_spec,
            pl.BlockSpec((1, 128), lambda i: (0, 0)),
        ],
        out_specs=out_specs,
        out_shape=out_shape,
    )(*parts, *slabs, inv, wlt, wrt, b)
    return out[0] if final else out


def kernel(x, edge_index, W1l, W1r, b1, W2l, W2r, b2, W3l, W3r, b3):
    e = edge_index.shape[1]
    src = edge_index[0].astype(jnp.int32)
    dst = edge_index[1].astype(jnp.int32)
    pad = jnp.full((_EPAD - e,), _N, jnp.int32)
    srcb = jnp.concatenate([src, pad]).reshape(_NBTOT, _BATCH)
    dstb = jnp.concatenate([dst, pad]).reshape(_NBTOT, _BATCH)

    x_pad = jnp.zeros((_NPAD, _C), jnp.float32)
    x_pad = x_pad.at[:_N, :26].set(x).at[:_N, 26].set(1.0)

    w1lt = jnp.zeros((_C, 128), jnp.float32).at[:26].set(W1l.T)
    w1rt = jnp.zeros((_C, 128), jnp.float32).at[:26].set(W1r.T)

    part1 = _sc_agg(x_pad, srcb, dstb)
    *slabs1, inv = _tc_layer1(part1, x_pad, w1lt, w1rt, b1.reshape(1, 128))

    parts2 = [_sc_agg(slabs1[c], srcb, dstb) for c in range(4)]
    slabs2 = _tc_layer(parts2, slabs1, inv, W2l.T, W2r.T,
                       b2.reshape(1, 128), final=False)

    parts3 = [_sc_agg(slabs2[c], srcb, dstb) for c in range(4)]
    return _tc_layer(parts3, slabs2, inv, W3l.T, W3r.T,
                     b3.reshape(1, 128), final=True)
